# baseline (device time: 32453 ns/iter reference)
import jax
import jax.numpy as jnp
from jax import lax
from jax.experimental import pallas as pl
from jax.experimental.pallas import tpu as pltpu

N_DEV = 32
NZ = 4
NQ = 8


def kernel(dy, W):
    m, k = dy.shape
    n = W.shape[0]
    zrows = m // NZ
    chunk = zrows // NQ

    def body(dy_ref, w_ref, out_ref, pbuf, zbuf, zacc_buf, pbufP, gbufP,
             blk, gbufZ, sendA, recvA, sendB, recvB, sendBp, recvBp,
             sendAp, recvAp):
        my = lax.axis_index("i")
        zi = my // NQ
        q = lax.rem(my, NQ)

        barrier = pltpu.get_barrier_semaphore()
        for dz in range(1, NZ):
            peer = lax.rem(zi + dz, NZ) * NQ + q
            pl.semaphore_signal(
                barrier, inc=1, device_id=(peer,),
                device_id_type=pl.DeviceIdType.MESH,
            )
        for dq in range(1, NQ):
            peer = zi * NQ + lax.rem(q + dq, NQ)
            pl.semaphore_signal(
                barrier, inc=1, device_id=(peer,),
                device_id_type=pl.DeviceIdType.MESH,
            )

        partial = lax.dot_general(
            dy_ref[...].astype(jnp.bfloat16),
            w_ref[...].astype(jnp.bfloat16),
            (((1,), (1,)), ((), ())),
            preferred_element_type=jnp.float32,
        )
        pbuf[...] = partial.astype(jnp.bfloat16).reshape(NZ, zrows, n)

        pl.semaphore_wait(barrier, NZ - 1 + NQ - 1)

        started = []

        for dz in range(1, NZ):
            tz = lax.rem(zi + dz, NZ)
            rdma = pltpu.make_async_remote_copy(
                src_ref=pbuf.at[tz],
                dst_ref=zbuf.at[zi],
                send_sem=sendA.at[tz],
                recv_sem=recvA.at[zi],
                device_id=(tz * NQ + q,),
                device_id_type=pl.DeviceIdType.MESH,
            )
            rdma.start()
            started.append(rdma)

        zbuf[zi, :, :] = pbuf[zi]
        for dz in range(1, NZ):
            sz = lax.rem(zi + dz, NZ)
            pltpu.make_async_remote_copy(
                src_ref=pbuf.at[sz],
                dst_ref=zbuf.at[sz],
                send_sem=sendA.at[sz],
                recv_sem=recvA.at[sz],
                device_id=(sz * NQ + q,),
                device_id_type=pl.DeviceIdType.MESH,
            ).wait_recv()
        zacc = jnp.sum(zbuf[...].astype(jnp.float32), axis=0)
        zacc_buf[...] = zacc.astype(jnp.bfloat16).reshape(NQ, chunk, n)


        for dq in range(1, NQ):
            tq = lax.rem(q + dq, NQ)
            rdma = pltpu.make_async_remote_copy(
                src_ref=zacc_buf.at[tq],
                dst_ref=pbufP.at[q],
                send_sem=sendB.at[tq],
                recv_sem=recvB.at[q],
                device_id=(zi * NQ + tq,),
                device_id_type=pl.DeviceIdType.MESH,
            )
            rdma.start()
            started.append(rdma)

        pbufP[q, :, :] = zacc_buf[q]
        for dq in range(1, NQ):
            sq = lax.rem(q + dq, NQ)
            pltpu.make_async_remote_copy(
                src_ref=zacc_buf.at[sq],
                dst_ref=pbufP.at[sq],
                send_sem=sendB.at[sq],
                recv_sem=recvB.at[sq],
                device_id=(zi * NQ + sq,),
                device_id_type=pl.DeviceIdType.MESH,
            ).wait_recv()
        acc16 = jnp.sum(pbufP[...].astype(jnp.float32), axis=0)

        out_ref[pl.ds((zi * NQ + q) * chunk, chunk), :] = acc16
        gbufP[q, :, :] = acc16.astype(jnp.bfloat16)


        for dq in range(1, NQ):
            tq = lax.rem(q + dq, NQ)
            rdma = pltpu.make_async_remote_copy(
                src_ref=gbufP.at[q],
                dst_ref=gbufP.at[q],
                send_sem=sendBp.at[tq],
                recv_sem=recvBp.at[q],
                device_id=(zi * NQ + tq,),
                device_id_type=pl.DeviceIdType.MESH,
            )
            rdma.start()
            started.append(rdma)

        for dq in range(1, NQ):
            sq = lax.rem(q + dq, NQ)
            pltpu.make_async_remote_copy(
                src_ref=gbufP.at[sq],
                dst_ref=gbufP.at[sq],
                send_sem=sendBp.at[sq],
                recv_sem=recvBp.at[sq],
                device_id=(zi * NQ + sq,),
                device_id_type=pl.DeviceIdType.MESH,
            ).wait_recv()
        blk[...] = gbufP[...].reshape(zrows, n)

        for dz in range(1, NZ):
            tz = lax.rem(zi + dz, NZ)
            rdma = pltpu.make_async_remote_copy(
                src_ref=blk,
                dst_ref=gbufZ.at[zi],
                send_sem=sendAp.at[tz],
                recv_sem=recvAp.at[zi],
                device_id=(tz * NQ + q,),
                device_id_type=pl.DeviceIdType.MESH,
            )
            rdma.start()
            started.append(rdma)

        gbufZ[zi, :, :] = blk[...]
        for dz in range(1, NZ):
            sz = lax.rem(zi + dz, NZ)
            pltpu.make_async_remote_copy(
                src_ref=blk,
                dst_ref=gbufZ.at[sz],
                send_sem=sendAp.at[sz],
                recv_sem=recvAp.at[sz],
                device_id=(sz * NQ + q,),
                device_id_type=pl.DeviceIdType.MESH,
            ).wait_recv()

        out_ref[...] = gbufZ[...].astype(jnp.float32).reshape(m, n)

        for rdma in started:
            rdma.wait_send()

    return pl.pallas_call(
        body,
        out_shape=jax.ShapeDtypeStruct((m, n), jnp.float32),
        in_specs=[
            pl.BlockSpec(memory_space=pltpu.VMEM),
            pl.BlockSpec(memory_space=pltpu.VMEM),
        ],
        out_specs=pl.BlockSpec(memory_space=pltpu.VMEM),
        scratch_shapes=[
            pltpu.VMEM((NZ, zrows, n), jnp.bfloat16),
            pltpu.VMEM((NZ, zrows, n), jnp.bfloat16),
            pltpu.VMEM((NQ, chunk, n), jnp.bfloat16),
            pltpu.VMEM((NQ, chunk, n), jnp.bfloat16),
            pltpu.VMEM((NQ, chunk, n), jnp.bfloat16),
            pltpu.VMEM((zrows, n), jnp.bfloat16),
            pltpu.VMEM((NZ, zrows, n), jnp.bfloat16),
            pltpu.SemaphoreType.DMA((NZ,)),
            pltpu.SemaphoreType.DMA((NZ,)),
            pltpu.SemaphoreType.DMA((NQ,)),
            pltpu.SemaphoreType.DMA((NQ,)),
            pltpu.SemaphoreType.DMA((NQ,)),
            pltpu.SemaphoreType.DMA((NQ,)),
            pltpu.SemaphoreType.DMA((NZ,)),
            pltpu.SemaphoreType.DMA((NZ,)),
        ],
        compiler_params=pltpu.CompilerParams(collective_id=0),
    )(dy, W)


# device time: 29718 ns/iter; 1.0920x vs baseline; 1.0920x over previous
import jax
import jax.numpy as jnp
from jax import lax
from jax.experimental import pallas as pl
from jax.experimental.pallas import tpu as pltpu

N_DEV = 32


def kernel(dy, W):
    m, k = dy.shape
    n = W.shape[0]
    chunk = m // N_DEV

    def body(dy_ref, w_ref, out_ref, pbuf, rs_buf, g_buf,
             send1, recv1, send2, recv2):
        my = lax.axis_index("i")

        barrier = pltpu.get_barrier_semaphore()
        for s in range(1, N_DEV):
            peer = lax.rem(my + s, N_DEV)
            pl.semaphore_signal(
                barrier, inc=1, device_id=(peer,),
                device_id_type=pl.DeviceIdType.MESH,
            )

        partial = lax.dot_general(
            dy_ref[...].astype(jnp.bfloat16),
            w_ref[...].astype(jnp.bfloat16),
            (((1,), (1,)), ((), ())),
            preferred_element_type=jnp.float32,
        )
        pbuf[...] = partial.astype(jnp.bfloat16).reshape(N_DEV, chunk, n)

        pl.semaphore_wait(barrier, N_DEV - 1)

        p1 = []
        for s in range(1, N_DEV):
            dst = lax.rem(my + s, N_DEV)
            rdma = pltpu.make_async_remote_copy(
                src_ref=pbuf.at[dst],
                dst_ref=rs_buf.at[my],
                send_sem=send1.at[s],
                recv_sem=recv1.at[my],
                device_id=(dst,),
                device_id_type=pl.DeviceIdType.MESH,
            )
            rdma.start()
            p1.append(rdma)

        rs_buf[my, :, :] = pbuf[my]

        acc = rs_buf[my].astype(jnp.float32)
        for g in range(0, N_DEV - 1, 8):
            srcs = []
            for s in range(g + 1, min(g + 9, N_DEV)):
                src = lax.rem(my - s + N_DEV, N_DEV)
                srcs.append(src)
                pltpu.make_async_remote_copy(
                    src_ref=pbuf.at[src],
                    dst_ref=rs_buf.at[src],
                    send_sem=send1.at[s],
                    recv_sem=recv1.at[src],
                    device_id=(src,),
                    device_id_type=pl.DeviceIdType.MESH,
                ).wait_recv()
            for src in srcs:
                acc = acc + rs_buf[src].astype(jnp.float32)

        g_buf[my, :, :] = acc.astype(jnp.bfloat16)

        p2 = []
        for s in range(1, N_DEV):
            dst = lax.rem(my + s, N_DEV)
            rdma = pltpu.make_async_remote_copy(
                src_ref=g_buf.at[my],
                dst_ref=g_buf.at[my],
                send_sem=send2.at[s],
                recv_sem=recv2.at[my],
                device_id=(dst,),
                device_id_type=pl.DeviceIdType.MESH,
            )
            rdma.start()
            p2.append(rdma)

        out_ref[pl.ds(my * chunk, chunk), :] = acc

        for s in range(1, N_DEV):
            src = lax.rem(my - s + N_DEV, N_DEV)
            pltpu.make_async_remote_copy(
                src_ref=g_buf.at[src],
                dst_ref=g_buf.at[src],
                send_sem=send2.at[s],
                recv_sem=recv2.at[src],
                device_id=(src,),
                device_id_type=pl.DeviceIdType.MESH,
            ).wait_recv()
            out_ref[pl.ds(src * chunk, chunk), :] = g_buf[src].astype(
                jnp.float32
            )

        for rdma in p1 + p2:
            rdma.wait_send()

    return pl.pallas_call(
        body,
        out_shape=jax.ShapeDtypeStruct((m, n), jnp.float32),
        in_specs=[
            pl.BlockSpec(memory_space=pltpu.VMEM),
            pl.BlockSpec(memory_space=pltpu.VMEM),
        ],
        out_specs=pl.BlockSpec(memory_space=pltpu.VMEM),
        scratch_shapes=[
            pltpu.VMEM((N_DEV, chunk, n), jnp.bfloat16),
            pltpu.VMEM((N_DEV, chunk, n), jnp.bfloat16),
            pltpu.VMEM((N_DEV, chunk, n), jnp.bfloat16),
            pltpu.SemaphoreType.DMA((N_DEV,)),
            pltpu.SemaphoreType.DMA((N_DEV,)),
            pltpu.SemaphoreType.DMA((N_DEV,)),
            pltpu.SemaphoreType.DMA((N_DEV,)),
        ],
        compiler_params=pltpu.CompilerParams(collective_id=0),
    )(dy, W)
